# baseline (speedup 1.0000x reference)
import functools

import jax
import jax.numpy as jnp
import numpy as np
from jax import lax
from jax.experimental import pallas as pl
from jax.experimental.pallas import tpu as pltpu
from jax.experimental.pallas import tpu_sc as plsc

CUT0, CUT1, CUT2 = 20000, 200000, 1000000
D0, D1, D2 = 1024, 256, 64
OUT_DIM = 1024

NC, NS, L = 2, 16, 16
NW = NC * NS

TBLK = 256

_Z = np.int32(0)


def _mesh():
  return plsc.VectorSubcoreMesh(core_axis_name="c", subcore_axis_name="s")


def _sc_band_body(ids_hbm, emb, g, ids_v, idx, buf0, buf1,
                  gs0, gs1, ws0, ws1, *, chunk, sub, lo, size):
  i32 = jnp.int32
  wid = lax.axis_index("s") * i32(NC) + lax.axis_index("c")
  base = wid * i32(chunk)
  pltpu.sync_copy(ids_hbm.at[pl.ds(base, chunk)], ids_v)

  def compute_idx(k, carry):
    o = k * i32(L)
    v = ids_v[pl.ds(o, L)]
    if lo:
      v = v - lo
    idx[pl.ds(o, L)] = jnp.clip(v, 0, size - 1)
    return carry

  lax.fori_loop(i32(0), i32(chunk // L), compute_idx, None)

  bufs = (buf0, buf1)
  gsems = (gs0, gs1)
  wsems = (ws0, ws1)
  nst = chunk // sub
  writes = [None, None]
  for j in range(nst):
    b = j % 2
    if writes[b] is not None:
      writes[b].wait()
    gcp = pltpu.async_copy(emb.at[idx.at[pl.ds(j * sub, sub)]],
                           bufs[b], gsems[b])
    gcp.wait()
    writes[b] = pltpu.async_copy(bufs[b], g.at[pl.ds(base + j * sub, sub)],
                                 wsems[b])
  for w in writes:
    if w is not None:
      w.wait()


def _sc_gather_band(ids, emb, *, sub, lo, size, untiled):
  n_tok = ids.shape[0]
  chunk = n_tok // NW
  d = emb.shape[1]
  params = {}
  if untiled:
    params["compiler_params"] = pltpu.CompilerParams(use_tc_tiling_on_sc=False)
  return pl.kernel(
      functools.partial(_sc_band_body, chunk=chunk, sub=sub, lo=lo, size=size),
      mesh=_mesh(),
      out_type=jax.ShapeDtypeStruct((n_tok, d), jnp.float32),
      scratch_types=[
          pltpu.VMEM((chunk,), jnp.int32),
          pltpu.VMEM((chunk,), jnp.int32),
          pltpu.VMEM((sub, d), jnp.float32),
          pltpu.VMEM((sub, d), jnp.float32),
          pltpu.SemaphoreType.DMA,
          pltpu.SemaphoreType.DMA,
          pltpu.SemaphoreType.DMA,
          pltpu.SemaphoreType.DMA,
      ],
      **params,
  )(ids, emb)


def _tc_matmul_body(ids_ref, g0_ref, g1_ref, g2_ref, w0t, w1t, w2t, out_ref):
  v = ids_ref[...]
  m0 = v < CUT0
  m1 = jnp.logical_and(v >= CUT0, v < CUT1)
  m2 = v >= CUT1
  x0 = jnp.where(m0, g0_ref[...], 0.0)
  x1 = jnp.where(m1, g1_ref[...], 0.0)
  x2 = jnp.where(m2, g2_ref[...], 0.0)
  acc = jnp.dot(x0, w0t[...], preferred_element_type=jnp.float32)
  acc += jnp.dot(x1, w1t[...], preferred_element_type=jnp.float32)
  acc += jnp.dot(x2, w2t[...], preferred_element_type=jnp.float32)
  out_ref[...] = acc


def _tc_matmul(ids2d, g0, g1, g2, w0t, w1t, w2t):
  n_tok = g0.shape[0]
  grid = (n_tok // TBLK,)
  return pl.pallas_call(
      _tc_matmul_body,
      grid=grid,
      in_specs=[
          pl.BlockSpec((TBLK, 1), lambda i: (i, _Z)),
          pl.BlockSpec((TBLK, D0), lambda i: (i, _Z)),
          pl.BlockSpec((TBLK, D1), lambda i: (i, _Z)),
          pl.BlockSpec((TBLK, D2), lambda i: (i, _Z)),
          pl.BlockSpec((D0, OUT_DIM), lambda i: (_Z, _Z)),
          pl.BlockSpec((D1, OUT_DIM), lambda i: (_Z, _Z)),
          pl.BlockSpec((D2, OUT_DIM), lambda i: (_Z, _Z)),
      ],
      out_specs=pl.BlockSpec((TBLK, OUT_DIM), lambda i: (i, _Z)),
      out_shape=jax.ShapeDtypeStruct((n_tok, OUT_DIM), jnp.float32),
  )(ids2d, g0, g1, g2, w0t, w1t, w2t)


def kernel(input, emb0, emb1, emb2, w0, w1, w2):
  ids = input.reshape(-1).astype(jnp.int32)
  g0 = _sc_gather_band(ids, emb0, sub=40, lo=0, size=CUT0, untiled=True)
  g1 = _sc_gather_band(ids, emb1, sub=200, lo=CUT0, size=CUT1 - CUT0,
                       untiled=True)
  g2 = _sc_gather_band(ids, emb2, sub=800, lo=CUT1, size=CUT2 - CUT1,
                       untiled=True)
  out = _tc_matmul(ids.reshape(-1, 1), g0, g1, g2,
                   w0.T, w1.T, w2.T)
  return out.reshape(input.shape + (OUT_DIM,))


# speedup vs baseline: 1.4419x; 1.4419x over previous
import functools

import jax
import jax.numpy as jnp
import numpy as np
from jax import lax
from jax.experimental import pallas as pl
from jax.experimental.pallas import tpu as pltpu
from jax.experimental.pallas import tpu_sc as plsc

CUT0, CUT1, CUT2 = 20000, 200000, 1000000
V0, V1, V2 = CUT0, CUT1 - CUT0, CUT2 - CUT1
D0, D1, D2 = 1024, 256, 64
OUT_DIM = 1024

NC, NS, L = 2, 16, 16
NW = NC * NS

SUB = 32
TBLK = 256

_Z = np.int32(0)


def _sc_gather_body(ids_hbm, emb0, emb1, emb2, g0, g1, g2,
                    ids_v, idx0, idx1, idx2,
                    b0a, b0b, b1a, b1b, b2a, b2b,
                    gsa, gsb, wsa, wsb, *, chunk):
  i32 = jnp.int32
  wid = lax.axis_index("s") * i32(NC) + lax.axis_index("c")
  base = wid * i32(chunk)
  pltpu.sync_copy(ids_hbm.at[pl.ds(base, chunk)], ids_v)

  nv = SUB // L

  def compute_idx(k, carry):
    j = k // i32(nv)
    m = k % i32(nv)
    v = ids_v[pl.ds(k * i32(L), L)]
    idx0[j, pl.ds(m * i32(L), L)] = jnp.clip(v, 0, V0 - 1)
    idx1[j, pl.ds(m * i32(L), L)] = jnp.clip(v - CUT0, 0, V1 - 1)
    idx2[j, pl.ds(m * i32(L), L)] = jnp.clip(v - CUT1, 0, V2 - 1)
    return carry

  lax.fori_loop(i32(0), i32(chunk // L), compute_idx, None)

  bufs = ((b0a, b1a, b2a), (b0b, b1b, b2b))
  gsems = (gsa, gsb)
  wsems = (wsa, wsb)
  embs = (emb0, emb1, emb2)
  idxs = (idx0, idx1, idx2)
  gs = (g0, g1, g2)
  nst = chunk // SUB
  writes = [None, None]
  for j in range(nst):
    p = j % 2
    if writes[p] is not None:
      for w in writes[p]:
        w.wait()
    jj = np.int32(j)
    gcps = [pltpu.async_copy(embs[t].at[idxs[t].at[jj]], bufs[p][t], gsems[p])
            for t in range(3)]
    for cp in gcps:
      cp.wait()
    writes[p] = [
        pltpu.async_copy(bufs[p][t], gs[t].at[pl.ds(base + j * SUB, SUB)],
                         wsems[p])
        for t in range(3)
    ]
  for p in range(2):
    if writes[p] is not None:
      for w in writes[p]:
        w.wait()


def _sc_gather(ids, emb0, emb1, emb2):
  n_tok = ids.shape[0]
  chunk = n_tok // NW
  nst = chunk // SUB
  return pl.kernel(
      functools.partial(_sc_gather_body, chunk=chunk),
      mesh=plsc.VectorSubcoreMesh(core_axis_name="c", subcore_axis_name="s"),
      compiler_params=pltpu.CompilerParams(use_tc_tiling_on_sc=False),
      out_type=[
          jax.ShapeDtypeStruct((n_tok, D0), jnp.float32),
          jax.ShapeDtypeStruct((n_tok, D1), jnp.float32),
          jax.ShapeDtypeStruct((n_tok, D2), jnp.float32),
      ],
      scratch_types=[
          pltpu.VMEM((chunk,), jnp.int32),
          pltpu.VMEM((nst, SUB), jnp.int32),
          pltpu.VMEM((nst, SUB), jnp.int32),
          pltpu.VMEM((nst, SUB), jnp.int32),
          pltpu.VMEM((SUB, D0), jnp.float32),
          pltpu.VMEM((SUB, D0), jnp.float32),
          pltpu.VMEM((SUB, D1), jnp.float32),
          pltpu.VMEM((SUB, D1), jnp.float32),
          pltpu.VMEM((SUB, D2), jnp.float32),
          pltpu.VMEM((SUB, D2), jnp.float32),
          pltpu.SemaphoreType.DMA,
          pltpu.SemaphoreType.DMA,
          pltpu.SemaphoreType.DMA,
          pltpu.SemaphoreType.DMA,
      ],
  )(ids, emb0, emb1, emb2)


def _tc_matmul_body(ids_ref, g0_ref, g1_ref, g2_ref, w0t, w1t, w2t, out_ref):
  v = ids_ref[...]
  m0 = v < CUT0
  m1 = jnp.logical_and(v >= CUT0, v < CUT1)
  m2 = v >= CUT1
  x0 = jnp.where(m0, g0_ref[...], 0.0)
  x1 = jnp.where(m1, g1_ref[...], 0.0)
  x2 = jnp.where(m2, g2_ref[...], 0.0)
  acc = jnp.dot(x0, w0t[...], preferred_element_type=jnp.float32)
  acc += jnp.dot(x1, w1t[...], preferred_element_type=jnp.float32)
  acc += jnp.dot(x2, w2t[...], preferred_element_type=jnp.float32)
  out_ref[...] = acc


def _tc_matmul(ids2d, g0, g1, g2, w0t, w1t, w2t):
  n_tok = ids2d.shape[0]
  grid = (n_tok // TBLK,)
  return pl.pallas_call(
      _tc_matmul_body,
      grid=grid,
      in_specs=[
          pl.BlockSpec((TBLK, 1), lambda i: (i, _Z)),
          pl.BlockSpec((TBLK, D0), lambda i: (i, _Z)),
          pl.BlockSpec((TBLK, D1), lambda i: (i, _Z)),
          pl.BlockSpec((TBLK, D2), lambda i: (i, _Z)),
          pl.BlockSpec((D0, OUT_DIM), lambda i: (_Z, _Z)),
          pl.BlockSpec((D1, OUT_DIM), lambda i: (_Z, _Z)),
          pl.BlockSpec((D2, OUT_DIM), lambda i: (_Z, _Z)),
      ],
      out_specs=pl.BlockSpec((TBLK, OUT_DIM), lambda i: (i, _Z)),
      out_shape=jax.ShapeDtypeStruct((n_tok, OUT_DIM), jnp.float32),
  )(ids2d, g0, g1, g2, w0t, w1t, w2t)


def kernel(input, emb0, emb1, emb2, w0, w1, w2):
  ids = input.reshape(-1).astype(jnp.int32)
  g0, g1, g2 = _sc_gather(ids, emb0, emb1, emb2)
  out = _tc_matmul(ids.reshape(-1, 1), g0, g1, g2,
                   w0.T, w1.T, w2.T)
  return out.reshape(input.shape + (OUT_DIM,))
